# baseline (speedup 1.0000x reference)
import functools

import jax
import jax.numpy as jnp
from jax import lax
from jax.experimental import pallas as pl
from jax.experimental.pallas import tpu as pltpu
from jax.experimental.pallas import tpu_sc as plsc

B = 16384
DIM = 32


@functools.lru_cache(maxsize=1)
def _build():
    info = plsc.get_sparse_core_info()
    NC, NS = info.num_cores, info.num_subcores
    NW = NC * NS
    bpw = B // NW
    mesh = plsc.VectorSubcoreMesh(core_axis_name="c", subcore_axis_name="s")

    @functools.partial(
        pl.kernel,
        mesh=mesh,
        compiler_params=pltpu.CompilerParams(use_tc_tiling_on_sc=False),
        out_type=(
            jax.ShapeDtypeStruct((B, 2 * DIM), jnp.float32),
            jax.ShapeDtypeStruct((B, 2 * DIM), jnp.float32),
        ),
        scratch_types=[
            pltpu.VMEM((bpw,), jnp.int32),
            pltpu.VMEM((bpw,), jnp.int32),
            pltpu.VMEM((bpw,), jnp.int32),
            pltpu.VMEM((bpw,), jnp.int32),
            pltpu.VMEM((bpw, DIM), jnp.float32),
            pltpu.VMEM((bpw, DIM), jnp.float32),
            pltpu.VMEM((bpw, DIM), jnp.float32),
            pltpu.VMEM((bpw, DIM), jnp.float32),
            pltpu.SemaphoreType.DMA,
        ],
    )
    def k(Wu, Wi, Wa, Wb, uid, iid, ca, cb, g_out, l_out,
          idx_u, idx_i, idx_a, idx_b, r_u, r_i, r_a, r_b, sem):
        wid = lax.axis_index("s") * NC + lax.axis_index("c")
        base = wid * bpw
        pltpu.sync_copy(uid.at[pl.ds(base, bpw)], idx_u)
        pltpu.sync_copy(iid.at[pl.ds(base, bpw)], idx_i)
        pltpu.sync_copy(ca.at[pl.ds(base, bpw)], idx_a)
        pltpu.sync_copy(cb.at[pl.ds(base, bpw)], idx_b)
        du = pltpu.async_copy(Wu.at[idx_u], r_u, sem)
        di = pltpu.async_copy(Wi.at[idx_i], r_i, sem)
        da = pltpu.async_copy(Wa.at[idx_a], r_a, sem)
        db = pltpu.async_copy(Wb.at[idx_b], r_b, sem)
        du.wait()
        pltpu.sync_copy(r_u, g_out.at[pl.ds(base, bpw), pl.ds(0, DIM)])
        di.wait()
        pltpu.sync_copy(r_i, g_out.at[pl.ds(base, bpw), pl.ds(DIM, DIM)])
        da.wait()
        pltpu.sync_copy(r_a, l_out.at[pl.ds(base, bpw), pl.ds(0, DIM)])
        db.wait()
        pltpu.sync_copy(r_b, l_out.at[pl.ds(base, bpw), pl.ds(DIM, DIM)])

    return k


def kernel(W_user, W_item, W_cat_a, W_cat_b, user_id, item_id, cat_a, cat_b):
    k = _build()
    return k(W_user, W_item, W_cat_a, W_cat_b,
             user_id.astype(jnp.int32), item_id.astype(jnp.int32),
             cat_a.astype(jnp.int32), cat_b.astype(jnp.int32))
